# baseline (speedup 1.0000x reference)
import functools

import jax
import jax.numpy as jnp
from jax import lax
from jax.experimental import pallas as pl
from jax.experimental.pallas import tpu as pltpu
from jax.experimental.pallas import tpu_sc as plsc

_N = 262144
_D = 512
_ATOM = 119
_PERIOD_MAP = {1: 1, 6: 2, 7: 2, 8: 2, 9: 2, 15: 3, 16: 3, 17: 3}

_NC = 2
_NS = 16
_NW = _NC * _NS
_BPW = _N // _NW
_C = 64
_NCHUNK = _BPW // _C


def _table_body(atom_ref, ptab_ref, w1a_ref, w1p_ref, b1_ref, w2_ref, b2_ref,
                out_ref):
    ptw = lax.dot_general(ptab_ref[...], w1p_ref[...], (((1,), (1,)), ((), ())),
                          preferred_element_type=jnp.float32)
    a = lax.broadcasted_iota(jnp.int32, (128, _D), 0)
    p = jnp.zeros((128, _D), jnp.int32)
    for num, per in _PERIOD_MAP.items():
        p = jnp.where(a == num, per, p)

    def _row(k):
        return jnp.broadcast_to(ptw[k:k + 1, :], (128, _D))

    pcon = jnp.where(p == 3, _row(3),
                     jnp.where(p == 2, _row(2),
                               jnp.where(p == 1, _row(1), _row(0))))
    acon = lax.dot_general(atom_ref[...], w1a_ref[...], (((1,), (1,)), ((), ())),
                           preferred_element_type=jnp.float32)
    h = jnp.maximum(acon + pcon + b1_ref[...], 0.0)
    out = lax.dot_general(h, w2_ref[...], (((1,), (1,)), ((), ())),
                          preferred_element_type=jnp.float32) + b2_ref[...]
    out_ref[...] = out


_table_call = pl.pallas_call(
    _table_body,
    out_shape=jax.ShapeDtypeStruct((128, _D), jnp.float32),
)

_TCB = 8192


def _tc_gather_body(x_ref, table_ref, out_ref):
    xb = x_ref[...]
    t = lax.broadcasted_iota(jnp.int32, (_TCB, 128), 1)
    onehot = (jnp.broadcast_to(xb, (_TCB, 128)) == t).astype(jnp.bfloat16)
    tab = table_ref[...]
    t_hi = tab.astype(jnp.bfloat16)
    t_lo = (tab - t_hi.astype(jnp.float32)).astype(jnp.bfloat16)
    dims = (((1,), (0,)), ((), ()))
    out_ref[...] = (
        lax.dot_general(onehot, t_hi, dims, preferred_element_type=jnp.float32)
        + lax.dot_general(onehot, t_lo, dims, preferred_element_type=jnp.float32))


def _tc_gather(table, x2d, n_rows):
    return pl.pallas_call(
        _tc_gather_body,
        grid=(n_rows // _TCB,),
        in_specs=[
            pl.BlockSpec((_TCB, 1), lambda i: (i, 0)),
            pl.BlockSpec((128, _D), lambda i: (0, 0)),
        ],
        out_specs=pl.BlockSpec((_TCB, _D), lambda i: (i, 0)),
        out_shape=jax.ShapeDtypeStruct((n_rows, _D), jnp.float32),
    )(x2d, table)


_SEG = 1024
_C2 = 32


@functools.cache
def _make_expand_call():
    mesh = plsc.VectorSubcoreMesh(core_axis_name="c", subcore_axis_name="s")
    npair = _BPW // _C2 // 2
    pairs_per_seg = _SEG // _C2 // 2

    @functools.partial(
        pl.kernel,
        out_type=jax.ShapeDtypeStruct((_N, _D), jnp.float32),
        mesh=mesh,
        scratch_types=[
            pltpu.VMEM((128, _D), jnp.float32),
            pltpu.VMEM((_C2, _D), jnp.float32),
            pltpu.VMEM((_C2, _D), jnp.float32),
            pltpu.VMEM((_BPW,), jnp.int32),
            pltpu.SemaphoreType.DMA,
            pltpu.SemaphoreType.DMA,
        ],
    )
    def _expand_call(table_hbm, idx_hbm, out_hbm, table_v, buf0, buf1, idx_s,
                     sem0, sem1):
        wid = lax.axis_index("s") * _NC + lax.axis_index("c")
        base = wid * _BPW
        pltpu.sync_copy(table_hbm, table_v)
        pltpu.sync_copy(idx_hbm.at[pl.ds(base, _BPW)], idx_s)
        bufs = (buf0, buf1)
        sems = (sem0, sem1)

        def fill(buf, ci):
            li0 = ci * _C2

            def rb_body(rb, carry):
                iv = idx_s[pl.ds(li0 + rb * 16, 16)]
                for j in range(16):
                    i = iv[j]
                    r = rb * 16 + j
                    for k in range(_D // 16):
                        buf[r, pl.ds(16 * k, 16)] = table_v[i,
                                                            pl.ds(16 * k, 16)]
                return carry

            lax.fori_loop(0, _C2 // 16, rb_body, 0)

        def pair_body(pi, carry):
            for b in range(2):
                ci = pi * 2 + b

                @pl.when(pi > 0)
                def _():
                    pltpu.make_async_copy(bufs[b],
                                          out_hbm.at[pl.ds(base, _C2)],
                                          sems[b]).wait()

                fill(bufs[b], ci)
                pltpu.async_copy(bufs[b],
                                 out_hbm.at[pl.ds(base + ci * _C2, _C2)],
                                 sems[b])
            return carry

        lax.fori_loop(0, npair, pair_body, 0)
        for b in range(2):
            pltpu.make_async_copy(bufs[b], out_hbm.at[pl.ds(base, _C2)],
                                  sems[b]).wait()

    return _expand_call


@functools.cache
def _make_gather_call(n_sc):
    bpw = n_sc // _NW
    nchunk = bpw // _C
    mesh = plsc.VectorSubcoreMesh(core_axis_name="c", subcore_axis_name="s")

    @functools.partial(
        pl.kernel,
        out_type=jax.ShapeDtypeStruct((n_sc, _D), jnp.float32),
        mesh=mesh,
        scratch_types=[
            pltpu.VMEM((bpw,), jnp.int32),
            pltpu.VMEM((_C, _D), jnp.float32),
            pltpu.SemaphoreType.DMA,
        ],
    )
    def _gather_call(table_hbm, idx_hbm, out_hbm, idx_v, rows_v, sem):
        wid = lax.axis_index("s") * _NC + lax.axis_index("c")
        base = wid * bpw
        pltpu.sync_copy(idx_hbm.at[pl.ds(base, bpw)], idx_v)

        def body(g, carry):
            start = g * _C
            pltpu.async_copy(table_hbm.at[idx_v.at[pl.ds(start, _C)]], rows_v,
                             sem).wait()
            pltpu.sync_copy(rows_v, out_hbm.at[pl.ds(base + start, _C)])
            return carry

        lax.fori_loop(0, nchunk, body, 0)

    return _gather_call


def kernel(x, atom_table, period_table, W1, b1, W2, b2):
    x = x.astype(jnp.int32)
    atom_pad = jnp.zeros((128, _D), jnp.float32).at[:_ATOM, :_D - 8].set(atom_table)
    w1a = jnp.concatenate([W1[:, :_D - 8], jnp.zeros((_D, 8), jnp.float32)], axis=1)
    w1p = jnp.concatenate([W1[:, _D - 8:], jnp.zeros((_D, 120), jnp.float32)], axis=1)
    ptab = jnp.concatenate([period_table, jnp.zeros((8, 120), jnp.float32)], axis=1)
    table = _table_call(atom_pad, ptab, w1a, w1p, b1.reshape(1, _D), W2,
                        b2.reshape(1, _D))
    return _make_expand_call()(table, x)


# speedup vs baseline: 1.1532x; 1.1532x over previous
import functools

import jax
import jax.numpy as jnp
from jax import lax
from jax.experimental import pallas as pl
from jax.experimental.pallas import tpu as pltpu
from jax.experimental.pallas import tpu_sc as plsc

_N = 262144
_D = 512
_ATOM = 119
_PERIOD_MAP = {1: 1, 6: 2, 7: 2, 8: 2, 9: 2, 15: 3, 16: 3, 17: 3}

_NC = 2
_NS = 16
_NW = _NC * _NS
_BPW = _N // _NW
_C = 64
_NCHUNK = _BPW // _C


def _table_body(atom_ref, ptab_ref, w1a_ref, w1p_ref, b1_ref, w2_ref, b2_ref,
                out_ref):
    ptw = lax.dot_general(ptab_ref[...], w1p_ref[...], (((1,), (1,)), ((), ())),
                          preferred_element_type=jnp.float32)
    a = lax.broadcasted_iota(jnp.int32, (128, _D), 0)
    p = jnp.zeros((128, _D), jnp.int32)
    for num, per in _PERIOD_MAP.items():
        p = jnp.where(a == num, per, p)

    def _row(k):
        return jnp.broadcast_to(ptw[k:k + 1, :], (128, _D))

    pcon = jnp.where(p == 3, _row(3),
                     jnp.where(p == 2, _row(2),
                               jnp.where(p == 1, _row(1), _row(0))))
    acon = lax.dot_general(atom_ref[...], w1a_ref[...], (((1,), (1,)), ((), ())),
                           preferred_element_type=jnp.float32)
    h = jnp.maximum(acon + pcon + b1_ref[...], 0.0)
    out = lax.dot_general(h, w2_ref[...], (((1,), (1,)), ((), ())),
                          preferred_element_type=jnp.float32) + b2_ref[...]
    out_ref[...] = out


_table_call = pl.pallas_call(
    _table_body,
    out_shape=jax.ShapeDtypeStruct((128, _D), jnp.float32),
)

_TCB = 8192


def _tc_gather_body(x_ref, table_ref, out_ref):
    xb = x_ref[...]
    t = lax.broadcasted_iota(jnp.int32, (_TCB, 128), 1)
    onehot = (jnp.broadcast_to(xb, (_TCB, 128)) == t).astype(jnp.bfloat16)
    tab = table_ref[...]
    t_hi = tab.astype(jnp.bfloat16)
    t_lo = (tab - t_hi.astype(jnp.float32)).astype(jnp.bfloat16)
    dims = (((1,), (0,)), ((), ()))
    out_ref[...] = (
        lax.dot_general(onehot, t_hi, dims, preferred_element_type=jnp.float32)
        + lax.dot_general(onehot, t_lo, dims, preferred_element_type=jnp.float32))


def _tc_gather(table, x2d, n_rows):
    return pl.pallas_call(
        _tc_gather_body,
        grid=(n_rows // _TCB,),
        in_specs=[
            pl.BlockSpec((_TCB, 1), lambda i: (i, 0)),
            pl.BlockSpec((128, _D), lambda i: (0, 0)),
        ],
        out_specs=pl.BlockSpec((_TCB, _D), lambda i: (i, 0)),
        out_shape=jax.ShapeDtypeStruct((n_rows, _D), jnp.float32),
    )(x2d, table)


_SEG = 1024
_C2 = 32


@functools.cache
def _make_expand_call():
    mesh = plsc.VectorSubcoreMesh(core_axis_name="c", subcore_axis_name="s")
    npair = _BPW // _C2 // 2
    pairs_per_seg = _SEG // _C2 // 2

    @functools.partial(
        pl.kernel,
        out_type=jax.ShapeDtypeStruct((_N, _D), jnp.float32),
        mesh=mesh,
        scratch_types=[
            pltpu.VMEM((128, _D), jnp.float32),
            pltpu.VMEM((_C2, _D), jnp.float32),
            pltpu.VMEM((_C2, _D), jnp.float32),
            pltpu.VMEM((_BPW,), jnp.int32),
            pltpu.SemaphoreType.DMA,
            pltpu.SemaphoreType.DMA,
        ],
    )
    def _expand_call(table_hbm, idx_hbm, out_hbm, table_v, buf0, buf1, idx_s,
                     sem0, sem1):
        wid = lax.axis_index("s") * _NC + lax.axis_index("c")
        base = wid * _BPW
        pltpu.sync_copy(table_hbm, table_v)
        pltpu.sync_copy(idx_hbm.at[pl.ds(base, _BPW)], idx_s)
        bufs = (buf0, buf1)
        sems = (sem0, sem1)

        def fill(buf, ci):
            li0 = ci * _C2

            def rb_body(rb, carry):
                iv = idx_s[pl.ds(li0 + rb * 16, 16)]
                for j in range(16):
                    i = iv[j]
                    r = rb * 16 + j
                    for k in range(_D // 16):
                        buf[r, pl.ds(16 * k, 16)] = table_v[i,
                                                            pl.ds(16 * k, 16)]
                return carry

            lax.fori_loop(0, _C2 // 16, rb_body, 0)

        def pair_body(pi, carry):
            for b in range(2):
                ci = pi * 2 + b

                @pl.when(pi > 0)
                def _():
                    pltpu.make_async_copy(bufs[b],
                                          out_hbm.at[pl.ds(base, _C2)],
                                          sems[b]).wait()

                fill(bufs[b], ci)
                pltpu.async_copy(bufs[b],
                                 out_hbm.at[pl.ds(base + ci * _C2, _C2)],
                                 sems[b])
            return carry

        lax.fori_loop(0, npair, pair_body, 0)
        for b in range(2):
            pltpu.make_async_copy(bufs[b], out_hbm.at[pl.ds(base, _C2)],
                                  sems[b]).wait()

    return _expand_call


_CH = 32
_NB = 4


@functools.cache
def _make_ring_call():
    nch = _BPW // _CH
    mesh = plsc.VectorSubcoreMesh(core_axis_name="c", subcore_axis_name="s")

    @functools.partial(
        pl.kernel,
        out_type=jax.ShapeDtypeStruct((_N, _D), jnp.float32),
        mesh=mesh,
        scratch_types=(
            [pltpu.VMEM((nch, _CH), jnp.int32)]
            + [pltpu.VMEM((_CH, _D), jnp.float32) for _ in range(_NB)]
            + [pltpu.SemaphoreType.DMA for _ in range(2 * _NB)]
        ),
    )
    def _ring_call(table_hbm, idx_hbm, out_hbm, idx_v, *rest):
        bufs = rest[:_NB]
        gsems = rest[_NB:2 * _NB]
        ssems = rest[2 * _NB:]
        wid = lax.axis_index("s") * _NC + lax.axis_index("c")
        base = wid * _BPW
        pltpu.sync_copy(idx_hbm.at[pl.ds(wid * nch, nch)], idx_v)

        def fire_gather(g, b):
            pltpu.async_copy(table_hbm.at[idx_v.at[g]], bufs[b], gsems[b])

        def wait_gather(b):
            pltpu.make_async_copy(table_hbm.at[idx_v.at[0]], bufs[b],
                                  gsems[b]).wait()

        def fire_scatter(g, b):
            pltpu.async_copy(bufs[b], out_hbm.at[pl.ds(base + g * _CH, _CH)],
                             ssems[b])

        def wait_scatter(b):
            pltpu.make_async_copy(bufs[b], out_hbm.at[pl.ds(base, _CH)],
                                  ssems[b]).wait()

        for g in range(_NB - 1):
            fire_gather(g, g)

        def quad_body(q, carry):
            for b in range(_NB):
                g = q * _NB + b
                wait_gather(b)
                fire_scatter(g, b)
                b3 = (b + _NB - 1) % _NB

                @pl.when(g + _NB - 1 < nch)
                def _():
                    @pl.when(g >= 1)
                    def _():
                        wait_scatter(b3)

                    fire_gather(g + _NB - 1, b3)
            return carry

        lax.fori_loop(0, nch // _NB, quad_body, 0)
        for b in range(_NB):
            wait_scatter(b)

    return _ring_call


@functools.cache
def _make_gather_call(n_sc):
    bpw = n_sc // _NW
    nchunk = bpw // _C
    mesh = plsc.VectorSubcoreMesh(core_axis_name="c", subcore_axis_name="s")

    @functools.partial(
        pl.kernel,
        out_type=jax.ShapeDtypeStruct((n_sc, _D), jnp.float32),
        mesh=mesh,
        scratch_types=[
            pltpu.VMEM((bpw,), jnp.int32),
            pltpu.VMEM((_C, _D), jnp.float32),
            pltpu.VMEM_SHARED((128, _D), jnp.float32),
            pltpu.SemaphoreType.DMA,
        ],
    )
    def _gather_call(table_hbm, idx_hbm, out_hbm, idx_v, rows_v, shared_t,
                     sem):
        wid = lax.axis_index("s") * _NC + lax.axis_index("c")
        base = wid * bpw

        @pl.when(lax.axis_index("s") == 0)
        def _():
            pltpu.sync_copy(table_hbm, shared_t)

        pltpu.sync_copy(idx_hbm.at[pl.ds(base, bpw)], idx_v)
        plsc.subcore_barrier()

        def body(g, carry):
            start = g * _C
            pltpu.async_copy(shared_t.at[idx_v.at[pl.ds(start, _C)]], rows_v,
                             sem).wait()
            pltpu.sync_copy(rows_v, out_hbm.at[pl.ds(base + start, _C)])
            return carry

        lax.fori_loop(0, nchunk, body, 0)

    return _gather_call


def kernel(x, atom_table, period_table, W1, b1, W2, b2):
    x = x.astype(jnp.int32)
    atom_pad = jnp.zeros((128, _D), jnp.float32).at[:_ATOM, :_D - 8].set(atom_table)
    w1a = jnp.concatenate([W1[:, :_D - 8], jnp.zeros((_D, 8), jnp.float32)], axis=1)
    w1p = jnp.concatenate([W1[:, _D - 8:], jnp.zeros((_D, 120), jnp.float32)], axis=1)
    ptab = jnp.concatenate([period_table, jnp.zeros((8, 120), jnp.float32)], axis=1)
    table = _table_call(atom_pad, ptab, w1a, w1p, b1.reshape(1, _D), W2,
                        b2.reshape(1, _D))
    return _make_ring_call()(table, x.reshape(_N // _CH, _CH))


# speedup vs baseline: 2.2434x; 1.9453x over previous
import functools

import jax
import jax.numpy as jnp
from jax import lax
from jax.experimental import pallas as pl
from jax.experimental.pallas import tpu as pltpu
from jax.experimental.pallas import tpu_sc as plsc

_N = 262144
_D = 512
_ATOM = 119
_PERIOD_MAP = {1: 1, 6: 2, 7: 2, 8: 2, 9: 2, 15: 3, 16: 3, 17: 3}

_NC = 2
_NS = 16
_NW = _NC * _NS

_NSC = 65536
_CH = 32
_NB = 4
_TCB = 8192


def _table_body(atom_ref, ptab_ref, w1a_ref, w1p_ref, b1_ref, w2_ref, b2_ref,
                out_ref):
    ptw = lax.dot_general(ptab_ref[...], w1p_ref[...], (((1,), (1,)), ((), ())),
                          preferred_element_type=jnp.float32)
    a = lax.broadcasted_iota(jnp.int32, (128, _D), 0)
    p = jnp.zeros((128, _D), jnp.int32)
    for num, per in _PERIOD_MAP.items():
        p = jnp.where(a == num, per, p)

    def _row(k):
        return jnp.broadcast_to(ptw[k:k + 1, :], (128, _D))

    pcon = jnp.where(p == 3, _row(3),
                     jnp.where(p == 2, _row(2),
                               jnp.where(p == 1, _row(1), _row(0))))
    acon = lax.dot_general(atom_ref[...], w1a_ref[...], (((1,), (1,)), ((), ())),
                           preferred_element_type=jnp.float32)
    h = jnp.maximum(acon + pcon + b1_ref[...], 0.0)
    out = lax.dot_general(h, w2_ref[...], (((1,), (1,)), ((), ())),
                          preferred_element_type=jnp.float32) + b2_ref[...]
    out_ref[...] = out


_table_call = pl.pallas_call(
    _table_body,
    out_shape=jax.ShapeDtypeStruct((128, _D), jnp.float32),
)


@functools.cache
def _make_ring_call():
    bpw = _NSC // _NW
    nch = bpw // _CH
    mesh = plsc.VectorSubcoreMesh(core_axis_name="c", subcore_axis_name="s")

    @functools.partial(
        pl.kernel,
        out_type=jax.ShapeDtypeStruct((_N, _D), jnp.float32),
        mesh=mesh,
        scratch_types=(
            [pltpu.VMEM((nch, _CH), jnp.int32)]
            + [pltpu.VMEM((_CH, _D), jnp.float32) for _ in range(_NB)]
            + [pltpu.SemaphoreType.DMA for _ in range(2 * _NB)]
        ),
    )
    def _ring_call(table_hbm, idx_hbm, out_hbm, idx_v, *rest):
        bufs = rest[:_NB]
        gsems = rest[_NB:2 * _NB]
        ssems = rest[2 * _NB:]
        wid = lax.axis_index("s") * _NC + lax.axis_index("c")
        base = wid * bpw
        pltpu.sync_copy(idx_hbm.at[pl.ds(wid * nch, nch)], idx_v)

        def fire_gather(g, b):
            pltpu.async_copy(table_hbm.at[idx_v.at[g]], bufs[b], gsems[b])

        def wait_gather(b):
            pltpu.make_async_copy(table_hbm.at[idx_v.at[0]], bufs[b],
                                  gsems[b]).wait()

        def fire_scatter(g, b):
            pltpu.async_copy(bufs[b], out_hbm.at[pl.ds(base + g * _CH, _CH)],
                             ssems[b])

        def wait_scatter(b):
            pltpu.make_async_copy(bufs[b], out_hbm.at[pl.ds(base, _CH)],
                                  ssems[b]).wait()

        for g in range(_NB - 1):
            fire_gather(g, g)

        def quad_body(q, carry):
            for b in range(_NB):
                g = q * _NB + b
                wait_gather(b)
                fire_scatter(g, b)
                b3 = (b + _NB - 1) % _NB

                @pl.when(g + _NB - 1 < nch)
                def _():
                    @pl.when(g >= 1)
                    def _():
                        wait_scatter(b3)

                    fire_gather(g + _NB - 1, b3)
            return carry

        lax.fori_loop(0, nch // _NB, quad_body, 0)
        for b in range(_NB):
            wait_scatter(b)

    return _ring_call


def _tc_tail_body(x_ref, table_ref, prev_ref, out_ref):
    del prev_ref
    xb = x_ref[...]
    t = lax.broadcasted_iota(jnp.int32, (_TCB, 128), 1)
    onehot = (jnp.broadcast_to(xb, (_TCB, 128)) == t).astype(jnp.float32)
    out_ref[...] = lax.dot_general(onehot, table_ref[...],
                                   (((1,), (0,)), ((), ())),
                                   preferred_element_type=jnp.float32)


def _tc_tail(table, x2d, prev):
    off = _NSC // _TCB
    nb = (_N - _NSC) // _TCB
    return pl.pallas_call(
        _tc_tail_body,
        grid=(nb,),
        in_specs=[
            pl.BlockSpec((_TCB, 1), lambda i: (i + off, 0)),
            pl.BlockSpec((128, _D), lambda i: (0, 0)),
            pl.BlockSpec(memory_space=pl.ANY),
        ],
        out_specs=pl.BlockSpec((_TCB, _D), lambda i: (i + off, 0)),
        out_shape=jax.ShapeDtypeStruct((_N, _D), jnp.float32),
        input_output_aliases={2: 0},
    )(x2d, table, prev)


def kernel(x, atom_table, period_table, W1, b1, W2, b2):
    x = x.astype(jnp.int32)
    atom_pad = jnp.zeros((128, _D), jnp.float32).at[:_ATOM, :_D - 8].set(atom_table)
    w1a = jnp.concatenate([W1[:, :_D - 8], jnp.zeros((_D, 8), jnp.float32)], axis=1)
    w1p = jnp.concatenate([W1[:, _D - 8:], jnp.zeros((_D, 120), jnp.float32)], axis=1)
    ptab = jnp.concatenate([period_table, jnp.zeros((8, 120), jnp.float32)], axis=1)
    table = _table_call(atom_pad, ptab, w1a, w1p, b1.reshape(1, _D), W2,
                        b2.reshape(1, _D))
    sc_out = _make_ring_call()(table, x.reshape(_N // _CH, _CH))
    return _tc_tail(table, x.reshape(_N, 1), sc_out)


# speedup vs baseline: 2.2783x; 1.0156x over previous
import functools

import jax
import jax.numpy as jnp
from jax import lax
from jax.experimental import pallas as pl
from jax.experimental.pallas import tpu as pltpu
from jax.experimental.pallas import tpu_sc as plsc

_N = 262144
_D = 512
_ATOM = 119
_PERIOD_MAP = {1: 1, 6: 2, 7: 2, 8: 2, 9: 2, 15: 3, 16: 3, 17: 3}

_NC = 2
_NS = 16
_NW = _NC * _NS

_NSC = 65536
_CH = 16
_NB = 8
_TCB = 8192


def _table_body(atom_ref, ptab_ref, w1a_ref, w1p_ref, b1_ref, w2_ref, b2_ref,
                out_ref):
    ptw = lax.dot_general(ptab_ref[...], w1p_ref[...], (((1,), (1,)), ((), ())),
                          preferred_element_type=jnp.float32)
    a = lax.broadcasted_iota(jnp.int32, (128, _D), 0)
    p = jnp.zeros((128, _D), jnp.int32)
    for num, per in _PERIOD_MAP.items():
        p = jnp.where(a == num, per, p)

    def _row(k):
        return jnp.broadcast_to(ptw[k:k + 1, :], (128, _D))

    pcon = jnp.where(p == 3, _row(3),
                     jnp.where(p == 2, _row(2),
                               jnp.where(p == 1, _row(1), _row(0))))
    acon = lax.dot_general(atom_ref[...], w1a_ref[...], (((1,), (1,)), ((), ())),
                           preferred_element_type=jnp.float32)
    h = jnp.maximum(acon + pcon + b1_ref[...], 0.0)
    out = lax.dot_general(h, w2_ref[...], (((1,), (1,)), ((), ())),
                          preferred_element_type=jnp.float32) + b2_ref[...]
    out_ref[...] = out


_table_call = pl.pallas_call(
    _table_body,
    out_shape=jax.ShapeDtypeStruct((128, _D), jnp.float32),
)


@functools.cache
def _make_ring_call():
    bpw = _NSC // _NW
    nch = bpw // _CH
    mesh = plsc.VectorSubcoreMesh(core_axis_name="c", subcore_axis_name="s")

    @functools.partial(
        pl.kernel,
        out_type=jax.ShapeDtypeStruct((_N, _D), jnp.float32),
        mesh=mesh,
        scratch_types=(
            [pltpu.VMEM((nch, _CH), jnp.int32)]
            + [pltpu.VMEM((_CH, _D), jnp.float32) for _ in range(_NB)]
            + [pltpu.SemaphoreType.DMA for _ in range(2 * _NB)]
        ),
    )
    def _ring_call(table_hbm, idx_hbm, out_hbm, idx_v, *rest):
        bufs = rest[:_NB]
        gsems = rest[_NB:2 * _NB]
        ssems = rest[2 * _NB:]
        wid = lax.axis_index("s") * _NC + lax.axis_index("c")
        base = wid * bpw
        pltpu.sync_copy(idx_hbm.at[pl.ds(wid * nch, nch)], idx_v)

        def fire_gather(g, b):
            pltpu.async_copy(table_hbm.at[idx_v.at[g]], bufs[b], gsems[b])

        def wait_gather(b):
            pltpu.make_async_copy(table_hbm.at[idx_v.at[0]], bufs[b],
                                  gsems[b]).wait()

        def fire_scatter(g, b):
            pltpu.async_copy(bufs[b], out_hbm.at[pl.ds(base + g * _CH, _CH)],
                             ssems[b])

        def wait_scatter(b):
            pltpu.make_async_copy(bufs[b], out_hbm.at[pl.ds(base, _CH)],
                                  ssems[b]).wait()

        for g in range(_NB - 1):
            fire_gather(g, g)

        def quad_body(q, carry):
            for b in range(_NB):
                g = q * _NB + b
                wait_gather(b)
                fire_scatter(g, b)
                b3 = (b + _NB - 1) % _NB

                @pl.when(g + _NB - 1 < nch)
                def _():
                    @pl.when(g >= 1)
                    def _():
                        wait_scatter(b3)

                    fire_gather(g + _NB - 1, b3)
            return carry

        lax.fori_loop(0, nch // _NB, quad_body, 0)
        for b in range(_NB):
            wait_scatter(b)

    return _ring_call


def _tc_tail_body(x_ref, table_ref, prev_ref, out_ref):
    del prev_ref
    xb = x_ref[...]
    t = lax.broadcasted_iota(jnp.int32, (_TCB, 128), 1)
    onehot = (jnp.broadcast_to(xb, (_TCB, 128)) == t).astype(jnp.float32)
    out_ref[...] = lax.dot_general(onehot, table_ref[...],
                                   (((1,), (0,)), ((), ())),
                                   preferred_element_type=jnp.float32)


def _tc_tail(table, x2d, prev):
    off = _NSC // _TCB
    nb = (_N - _NSC) // _TCB
    return pl.pallas_call(
        _tc_tail_body,
        grid=(nb,),
        in_specs=[
            pl.BlockSpec((_TCB, 1), lambda i: (i + off, 0)),
            pl.BlockSpec((128, _D), lambda i: (0, 0)),
            pl.BlockSpec(memory_space=pl.ANY),
        ],
        out_specs=pl.BlockSpec((_TCB, _D), lambda i: (i + off, 0)),
        out_shape=jax.ShapeDtypeStruct((_N, _D), jnp.float32),
        input_output_aliases={2: 0},
    )(x2d, table, prev)


def kernel(x, atom_table, period_table, W1, b1, W2, b2):
    x = x.astype(jnp.int32)
    atom_pad = jnp.zeros((128, _D), jnp.float32).at[:_ATOM, :_D - 8].set(atom_table)
    w1a = jnp.concatenate([W1[:, :_D - 8], jnp.zeros((_D, 8), jnp.float32)], axis=1)
    w1p = jnp.concatenate([W1[:, _D - 8:], jnp.zeros((_D, 120), jnp.float32)], axis=1)
    ptab = jnp.concatenate([period_table, jnp.zeros((8, 120), jnp.float32)], axis=1)
    table = _table_call(atom_pad, ptab, w1a, w1p, b1.reshape(1, _D), W2,
                        b2.reshape(1, _D))
    sc_out = _make_ring_call()(table, x.reshape(_N // _CH, _CH))
    return _tc_tail(table, x.reshape(_N, 1), sc_out)
